# baseline (speedup 1.0000x reference)
import jax
import jax.numpy as jnp
from jax.experimental import pallas as pl
from jax.experimental.pallas import tpu as pltpu


def _body(x_ref, p_ref, o_ref):
    o_ref[...] = x_ref[...] + p_ref[...]


def _part(x, pos_table, b0, nb):
    b, s, d = x.shape
    sb = 2048
    grid = (s // sb, nb)
    return pl.pallas_call(
        _body,
        grid=grid,
        in_specs=[
            pl.BlockSpec((1, sb, d), lambda i, j: (b0 + j, i, 0)),
            pl.BlockSpec((sb, d), lambda i, j: (i, 0)),
        ],
        out_specs=pl.BlockSpec((1, sb, d), lambda i, j: (j, i, 0)),
        out_shape=jax.ShapeDtypeStruct((nb, s, d), x.dtype),
        compiler_params=pltpu.CompilerParams(
            dimension_semantics=("parallel", "parallel"),
        ),
    )(x, pos_table)


def kernel(x, pos_table):
    lo = _part(x, pos_table, 0, 2)
    hi = _part(x, pos_table, 2, 2)
    return jnp.concatenate([lo, hi], axis=0)
